# baseline (speedup 1.0000x reference)
import dataclasses
import functools

import jax
import jax.numpy as jnp
import numpy as np
from jax import lax
from jax.experimental import pallas as pl
from jax.experimental.pallas import tpu as pltpu
from jax.experimental.pallas import tpu_sc as plsc

BATCH = 1024
MPD = 50
EMB = 128
LANES = 16
NC, NS = 2, 16
NW = NC * NS
DOCS_PER_W = BATCH // NW
PAIR = 2
PAIRS_PER_W = DOCS_PER_W // PAIR
NBUF = 4

_PERM = np.empty(EMB, dtype=np.int32)
for _c in range(EMB // 32):
    _k = np.arange(16)
    _PERM[32 * _c + 2 * _k] = 32 * _c + _k
    _PERM[32 * _c + 2 * _k + 1] = 32 * _c + 16 + _k


def _sc_embedding_bag(idx, table, bias):
    mesh = plsc.VectorSubcoreMesh(core_axis_name="c", subcore_axis_name="s")
    cp = pltpu.CompilerParams()
    if "needs_layout_passes" in pltpu.CompilerParams.__dataclass_fields__:
        cp = dataclasses.replace(cp, needs_layout_passes=False)
    if "use_tc_tiling_on_sc" in pltpu.CompilerParams.__dataclass_fields__:
        cp = dataclasses.replace(cp, use_tc_tiling_on_sc=False)

    @functools.partial(
        pl.kernel,
        out_type=jax.ShapeDtypeStruct((BATCH, EMB), jnp.float32),
        mesh=mesh,
        compiler_params=cp,
        scratch_types=[
            pltpu.VMEM((PAIRS_PER_W, PAIR * MPD), jnp.int32),
        ]
        + [pltpu.VMEM((PAIR * MPD, EMB // 2), jnp.int32)] * NBUF
        + [
            pltpu.VMEM((DOCS_PER_W, EMB), jnp.float32),
            pltpu.VMEM((EMB,), jnp.float32),
        ]
        + [pltpu.SemaphoreType.DMA] * NBUF,
    )
    def kern(idx_hbm, tab_hbm, b_hbm, out_hbm, idx_v, *rest):
        rows_bufs = rest[:NBUF]
        out_v, bias_v = rest[NBUF], rest[NBUF + 1]
        sems = rest[NBUF + 2:]
        wid = lax.axis_index("s") * NC + lax.axis_index("c")
        base = wid * DOCS_PER_W
        pltpu.sync_copy(b_hbm, bias_v)
        pltpu.sync_copy(idx_hbm.at[pl.ds(wid * PAIRS_PER_W, PAIRS_PER_W)], idx_v)

        for j in range(NBUF):
            pltpu.async_copy(tab_hbm.at[idx_v.at[j]], rows_bufs[j], sems[j])

        @pl.loop(0, PAIRS_PER_W, step=NBUF)
        def _pair(p0):
            for j in range(NBUF):
                p = p0 + j
                rows = rows_bufs[j]
                pltpu.make_async_copy(
                    tab_hbm.at[idx_v.at[p]], rows, sems[j]).wait()
                for sub in range(PAIR):
                    accs0 = tuple(bias_v[pl.ds(c * LANES, LANES)]
                                  for c in range(EMB // LANES))

                    def body(r, accs):
                        new = list(accs)
                        for g in range(EMB // 32):
                            x = plsc.bitcast(
                                rows[r, pl.ds(LANES * g, LANES)], jnp.bfloat16)
                            a, b2 = plsc.unpack(
                                x, format=plsc.PackFormat.INTERLEAVED)
                            new[2 * g] = new[2 * g] + a
                            new[2 * g + 1] = new[2 * g + 1] + b2
                        return tuple(new)

                    accs = plsc.parallel_loop(
                        sub * MPD, (sub + 1) * MPD, 1, unroll=5,
                        carry=accs0)(body)
                    d = p * PAIR + sub
                    for c in range(EMB // LANES):
                        out_v[d, pl.ds(c * LANES, LANES)] = jnp.maximum(
                            accs[c], 0.0)

                @pl.when(p + NBUF < PAIRS_PER_W)
                def _():
                    pltpu.async_copy(
                        tab_hbm.at[idx_v.at[p + NBUF]], rows, sems[j])

        pltpu.sync_copy(out_v, out_hbm.at[pl.ds(base, DOCS_PER_W)])

    return kern(idx, table, bias)


def kernel(document_mention_indices, W, b):
    idx = document_mention_indices.astype(jnp.int32).reshape(
        BATCH // PAIR, PAIR * MPD)
    table = W.T[:, _PERM].astype(jnp.bfloat16)
    table = jax.lax.bitcast_convert_type(
        table.reshape(table.shape[0], EMB // 2, 2), jnp.int32)
    return _sc_embedding_bag(idx, table, b)


# speedup vs baseline: 1.0006x; 1.0006x over previous
import dataclasses
import functools

import jax
import jax.numpy as jnp
import numpy as np
from jax import lax
from jax.experimental import pallas as pl
from jax.experimental.pallas import tpu as pltpu
from jax.experimental.pallas import tpu_sc as plsc

BATCH = 1024
MPD = 50
EMB = 128
LANES = 16
NC, NS = 2, 16
NW = NC * NS
DOCS_PER_W = BATCH // NW
PAIR = 2
PAIRS_PER_W = DOCS_PER_W // PAIR
NBUF = 4

_PERM = np.empty(EMB, dtype=np.int32)
for _c in range(EMB // 32):
    _k = np.arange(16)
    _PERM[32 * _c + 2 * _k] = 32 * _c + _k
    _PERM[32 * _c + 2 * _k + 1] = 32 * _c + 16 + _k


def _sc_embedding_bag(idx, table, bias):
    mesh = plsc.VectorSubcoreMesh(core_axis_name="c", subcore_axis_name="s")
    cp = pltpu.CompilerParams()
    if "needs_layout_passes" in pltpu.CompilerParams.__dataclass_fields__:
        cp = dataclasses.replace(cp, needs_layout_passes=False)
    if "use_tc_tiling_on_sc" in pltpu.CompilerParams.__dataclass_fields__:
        cp = dataclasses.replace(cp, use_tc_tiling_on_sc=False)

    @functools.partial(
        pl.kernel,
        out_type=jax.ShapeDtypeStruct((BATCH, EMB), jnp.float32),
        mesh=mesh,
        compiler_params=cp,
        scratch_types=[
            pltpu.VMEM((PAIRS_PER_W, PAIR * MPD), jnp.int32),
        ]
        + [pltpu.VMEM((PAIR * MPD, EMB // 2), jnp.int32)] * NBUF
        + [
            pltpu.VMEM((DOCS_PER_W, EMB), jnp.float32),
            pltpu.VMEM((EMB,), jnp.float32),
        ]
        + [pltpu.SemaphoreType.DMA] * NBUF,
    )
    def kern(idx_hbm, tab_hbm, b_hbm, out_hbm, idx_v, *rest):
        rows_bufs = rest[:NBUF]
        out_v, bias_v = rest[NBUF], rest[NBUF + 1]
        sems = rest[NBUF + 2:]
        wid = lax.axis_index("s") * NC + lax.axis_index("c")
        base = wid * DOCS_PER_W
        pltpu.sync_copy(b_hbm, bias_v)
        pltpu.sync_copy(idx_hbm.at[pl.ds(wid * PAIRS_PER_W, PAIRS_PER_W)], idx_v)

        for j in range(NBUF):
            pltpu.async_copy(tab_hbm.at[idx_v.at[j]], rows_bufs[j], sems[j])

        @pl.loop(0, PAIRS_PER_W, step=NBUF)
        def _pair(p0):
            for j in range(NBUF):
                p = p0 + j
                rows = rows_bufs[j]
                pltpu.make_async_copy(
                    tab_hbm.at[idx_v.at[p]], rows, sems[j]).wait()
                for sub in range(PAIR):
                    accs0 = tuple(bias_v[pl.ds(c * LANES, LANES)]
                                  for c in range(EMB // LANES))

                    def body(r, accs):
                        new = list(accs)
                        for g in range(EMB // 32):
                            x = plsc.bitcast(
                                rows[r, pl.ds(LANES * g, LANES)], jnp.bfloat16)
                            a, b2 = plsc.unpack(
                                x, format=plsc.PackFormat.INTERLEAVED)
                            new[2 * g] = new[2 * g] + a
                            new[2 * g + 1] = new[2 * g + 1] + b2
                        return tuple(new)

                    accs = plsc.parallel_loop(
                        sub * MPD, (sub + 1) * MPD, 1, unroll=5,
                        carry=accs0)(body)
                    d = p * PAIR + sub
                    for c in range(EMB // LANES):
                        out_v[d, pl.ds(c * LANES, LANES)] = jnp.maximum(
                            accs[c], 0.0)

                @pl.when(p + NBUF < PAIRS_PER_W)
                def _():
                    pltpu.async_copy(
                        tab_hbm.at[idx_v.at[p + NBUF]], rows, sems[j])

        pltpu.sync_copy(out_v, out_hbm.at[pl.ds(base, DOCS_PER_W)])

    return kern(idx, table, bias)


def kernel(document_mention_indices, W, b):
    idx = document_mention_indices.astype(jnp.int32).reshape(
        BATCH // PAIR, PAIR * MPD)
    table = W[_PERM, :].T.astype(jnp.bfloat16)
    table = jax.lax.bitcast_convert_type(
        table.reshape(table.shape[0], EMB // 2, 2), jnp.int32)
    return _sc_embedding_bag(idx, table, b)


# speedup vs baseline: 2.0060x; 2.0047x over previous
import dataclasses
import functools

import jax
import jax.numpy as jnp
import numpy as np
from jax import lax
from jax.experimental import pallas as pl
from jax.experimental.pallas import tpu as pltpu
from jax.experimental.pallas import tpu_sc as plsc

BATCH = 1024
MPD = 50
EMB = 128
LANES = 16
NC, NS = 2, 16
NW = NC * NS
DOCS_PER_W = BATCH // NW
PAIR = 2
PAIRS_PER_W = DOCS_PER_W // PAIR
NBUF = 4

_PLO = np.empty(EMB // 2, dtype=np.int32)
_PHI = np.empty(EMB // 2, dtype=np.int32)
for _g in range(EMB // 32):
    _k = np.arange(16)
    _PLO[16 * _g + _k] = 32 * _g + _k
    _PHI[16 * _g + _k] = 32 * _g + 16 + _k


def _bf16_bits(u):
    return (u + 0x7FFF + ((u >> 16) & 1)) >> 16


def _sc_embedding_bag(idx, table, bias):
    mesh = plsc.VectorSubcoreMesh(core_axis_name="c", subcore_axis_name="s")
    cp = pltpu.CompilerParams()
    if "needs_layout_passes" in pltpu.CompilerParams.__dataclass_fields__:
        cp = dataclasses.replace(cp, needs_layout_passes=False)
    if "use_tc_tiling_on_sc" in pltpu.CompilerParams.__dataclass_fields__:
        cp = dataclasses.replace(cp, use_tc_tiling_on_sc=False)

    @functools.partial(
        pl.kernel,
        out_type=jax.ShapeDtypeStruct((BATCH, EMB), jnp.float32),
        mesh=mesh,
        compiler_params=cp,
        scratch_types=[
            pltpu.VMEM((PAIRS_PER_W, PAIR * MPD), jnp.int32),
        ]
        + [pltpu.VMEM((PAIR * MPD, EMB // 2), jnp.int32)] * NBUF
        + [
            pltpu.VMEM((DOCS_PER_W, EMB), jnp.float32),
            pltpu.VMEM((EMB,), jnp.float32),
        ]
        + [pltpu.SemaphoreType.DMA] * NBUF,
    )
    def kern(idx_hbm, tab_hbm, b_hbm, out_hbm, idx_v, *rest):
        rows_bufs = rest[:NBUF]
        out_v, bias_v = rest[NBUF], rest[NBUF + 1]
        sems = rest[NBUF + 2:]
        wid = lax.axis_index("s") * NC + lax.axis_index("c")
        base = wid * DOCS_PER_W
        pltpu.sync_copy(b_hbm, bias_v)
        pltpu.sync_copy(idx_hbm.at[pl.ds(wid * PAIRS_PER_W, PAIRS_PER_W)], idx_v)

        for j in range(NBUF):
            pltpu.async_copy(tab_hbm.at[idx_v.at[j]], rows_bufs[j], sems[j])

        @pl.loop(0, PAIRS_PER_W, step=NBUF)
        def _pair(p0):
            for j in range(NBUF):
                p = p0 + j
                rows = rows_bufs[j]
                pltpu.make_async_copy(
                    tab_hbm.at[idx_v.at[p]], rows, sems[j]).wait()
                for sub in range(PAIR):
                    accs0 = tuple(bias_v[pl.ds(c * LANES, LANES)]
                                  for c in range(EMB // LANES))

                    def body(r, accs):
                        new = list(accs)
                        for g in range(EMB // 32):
                            x = plsc.bitcast(
                                rows[r, pl.ds(LANES * g, LANES)], jnp.bfloat16)
                            a, b2 = plsc.unpack(
                                x, format=plsc.PackFormat.INTERLEAVED)
                            new[2 * g] = new[2 * g] + a
                            new[2 * g + 1] = new[2 * g + 1] + b2
                        return tuple(new)

                    accs = plsc.parallel_loop(
                        sub * MPD, (sub + 1) * MPD, 1, unroll=5,
                        carry=accs0)(body)
                    d = p * PAIR + sub
                    for c in range(EMB // LANES):
                        out_v[d, pl.ds(c * LANES, LANES)] = jnp.maximum(
                            accs[c], 0.0)

                @pl.when(p + NBUF < PAIRS_PER_W)
                def _():
                    pltpu.async_copy(
                        tab_hbm.at[idx_v.at[p + NBUF]], rows, sems[j])

        pltpu.sync_copy(out_v, out_hbm.at[pl.ds(base, DOCS_PER_W)])

    return kern(idx, table, bias)


def kernel(document_mention_indices, W, b):
    idx = document_mention_indices.astype(jnp.int32).reshape(
        BATCH // PAIR, PAIR * MPD)
    u = jax.lax.bitcast_convert_type(W, jnp.uint32)
    word = _bf16_bits(u[_PLO, :]) | (_bf16_bits(u[_PHI, :]) << 16)
    table = jax.lax.bitcast_convert_type(word, jnp.int32).T
    return _sc_embedding_bag(idx, table, b)


# speedup vs baseline: 2.9775x; 1.4843x over previous
import dataclasses
import functools

import jax
import jax.numpy as jnp
import numpy as np
from jax import lax
from jax.experimental import pallas as pl
from jax.experimental.pallas import tpu as pltpu
from jax.experimental.pallas import tpu_sc as plsc

BATCH = 1024
MPD = 50
EMB = 128
LANES = 16
NC, NS = 2, 16
NW = NC * NS
DOCS_PER_W = BATCH // NW
PAIR = 2
PAIRS_PER_W = DOCS_PER_W // PAIR
NBUF = 4

_PLO = np.empty(EMB // 2, dtype=np.int32)
_PHI = np.empty(EMB // 2, dtype=np.int32)
for _g in range(EMB // 32):
    _k = np.arange(16)
    _PLO[16 * _g + _k] = 32 * _g + _k
    _PHI[16 * _g + _k] = 32 * _g + 16 + _k


def _bf16_bits(u):
    return (u + 0x7FFF + ((u >> 16) & 1)) >> 16


def _sc_embedding_bag(idx, table, bias):
    mesh = plsc.VectorSubcoreMesh(core_axis_name="c", subcore_axis_name="s")
    cp = pltpu.CompilerParams()
    if "needs_layout_passes" in pltpu.CompilerParams.__dataclass_fields__:
        cp = dataclasses.replace(cp, needs_layout_passes=False)
    if "use_tc_tiling_on_sc" in pltpu.CompilerParams.__dataclass_fields__:
        cp = dataclasses.replace(cp, use_tc_tiling_on_sc=False)

    @functools.partial(
        pl.kernel,
        out_type=jax.ShapeDtypeStruct((BATCH, EMB), jnp.float32),
        mesh=mesh,
        compiler_params=cp,
        scratch_types=[
            pltpu.VMEM((PAIRS_PER_W, PAIR * MPD), jnp.int32),
        ]
        + [pltpu.VMEM((PAIR * MPD, EMB // 2), jnp.int32)] * NBUF
        + [
            pltpu.VMEM((DOCS_PER_W, EMB), jnp.float32),
            pltpu.VMEM((EMB,), jnp.float32),
        ]
        + [pltpu.SemaphoreType.DMA] * NBUF,
    )
    def kern(idx_hbm, tab_hbm, b_hbm, out_hbm, idx_v, *rest):
        rows_bufs = rest[:NBUF]
        out_v, bias_v = rest[NBUF], rest[NBUF + 1]
        sems = rest[NBUF + 2:]
        wid = lax.axis_index("s") * NC + lax.axis_index("c")
        base = wid * DOCS_PER_W
        pltpu.sync_copy(b_hbm, bias_v)
        pltpu.sync_copy(idx_hbm.at[pl.ds(wid * PAIRS_PER_W, PAIRS_PER_W)], idx_v)

        for j in range(NBUF):
            pltpu.async_copy(tab_hbm.at[idx_v.at[j]], rows_bufs[j], sems[j])

        @pl.loop(0, PAIRS_PER_W, step=NBUF)
        def _pair(p0):
            for j in range(NBUF):
                p = p0 + j
                rows = rows_bufs[j]
                pltpu.make_async_copy(
                    tab_hbm.at[idx_v.at[p]], rows, sems[j]).wait()
                for sub in range(PAIR):
                    accs0 = tuple(bias_v[pl.ds(c * LANES, LANES)]
                                  for c in range(EMB // LANES))

                    def body(r, accs):
                        new = list(accs)
                        for g in range(EMB // 32):
                            x = plsc.bitcast(
                                rows[r, pl.ds(LANES * g, LANES)], jnp.bfloat16)
                            a, b2 = plsc.unpack(
                                x, format=plsc.PackFormat.INTERLEAVED)
                            new[2 * g] = new[2 * g] + a
                            new[2 * g + 1] = new[2 * g + 1] + b2
                        return tuple(new)

                    accs = plsc.parallel_loop(
                        sub * MPD, (sub + 1) * MPD, 1, unroll=5,
                        carry=accs0)(body)
                    d = p * PAIR + sub
                    for c in range(EMB // LANES):
                        out_v[d, pl.ds(c * LANES, LANES)] = jnp.maximum(
                            accs[c], 0.0)

                @pl.when(p + NBUF < PAIRS_PER_W)
                def _():
                    pltpu.async_copy(
                        tab_hbm.at[idx_v.at[p + NBUF]], rows, sems[j])

        pltpu.sync_copy(out_v, out_hbm.at[pl.ds(base, DOCS_PER_W)])

    return kern(idx, table, bias)


def kernel(document_mention_indices, W, b):
    idx = document_mention_indices.astype(jnp.int32).reshape(
        BATCH // PAIR, PAIR * MPD)
    u = jax.lax.bitcast_convert_type(W, jnp.uint32).reshape(
        EMB // 32, 2, 16, W.shape[1])
    word = _bf16_bits(u[:, 0]) | (_bf16_bits(u[:, 1]) << 16)
    table = jax.lax.bitcast_convert_type(
        word.reshape(EMB // 2, W.shape[1]), jnp.int32).T
    return _sc_embedding_bag(idx, table, b)


# speedup vs baseline: 3.0174x; 1.0134x over previous
import dataclasses
import functools

import jax
import jax.numpy as jnp
import numpy as np
from jax import lax
from jax.experimental import pallas as pl
from jax.experimental.pallas import tpu as pltpu
from jax.experimental.pallas import tpu_sc as plsc

BATCH = 1024
MPD = 50
EMB = 128
LANES = 16
NC, NS = 2, 16
NW = NC * NS
DOCS_PER_W = BATCH // NW
PAIR = 2
PAIRS_PER_W = DOCS_PER_W // PAIR
NBUF = 4

_PLO = np.empty(EMB // 2, dtype=np.int32)
_PHI = np.empty(EMB // 2, dtype=np.int32)
for _g in range(EMB // 32):
    _k = np.arange(16)
    _PLO[16 * _g + _k] = 32 * _g + _k
    _PHI[16 * _g + _k] = 32 * _g + 16 + _k


def _bf16_bits(u):
    return (u + 0x7FFF + ((u >> 16) & 1)) >> 16


def _sc_embedding_bag(idx, table, bias):
    mesh = plsc.VectorSubcoreMesh(core_axis_name="c", subcore_axis_name="s")
    cp = pltpu.CompilerParams()
    if "needs_layout_passes" in pltpu.CompilerParams.__dataclass_fields__:
        cp = dataclasses.replace(cp, needs_layout_passes=False)
    if "use_tc_tiling_on_sc" in pltpu.CompilerParams.__dataclass_fields__:
        cp = dataclasses.replace(cp, use_tc_tiling_on_sc=False)

    @functools.partial(
        pl.kernel,
        out_type=jax.ShapeDtypeStruct((BATCH, EMB), jnp.float32),
        mesh=mesh,
        compiler_params=cp,
        scratch_types=[
            pltpu.VMEM((PAIRS_PER_W, PAIR * MPD), jnp.int32),
        ]
        + [pltpu.VMEM((PAIR * MPD, EMB // 2), jnp.int32)] * NBUF
        + [
            pltpu.VMEM((DOCS_PER_W, EMB), jnp.float32),
            pltpu.VMEM((EMB,), jnp.float32),
        ]
        + [pltpu.SemaphoreType.DMA] * NBUF,
    )
    def kern(idx_hbm, tab_hbm, b_hbm, out_hbm, idx_v, *rest):
        rows_bufs = rest[:NBUF]
        out_v, bias_v = rest[NBUF], rest[NBUF + 1]
        sems = rest[NBUF + 2:]
        wid = lax.axis_index("s") * NC + lax.axis_index("c")
        base = wid * DOCS_PER_W
        pltpu.sync_copy(b_hbm, bias_v)
        pltpu.sync_copy(idx_hbm.at[pl.ds(wid * PAIRS_PER_W, PAIRS_PER_W)], idx_v)

        for j in range(NBUF):
            pltpu.async_copy(tab_hbm.at[idx_v.at[j]], rows_bufs[j], sems[j])

        @pl.loop(0, PAIRS_PER_W, step=NBUF)
        def _pair(p0):
            for j in range(NBUF):
                p = p0 + j
                rows = rows_bufs[j]
                pltpu.make_async_copy(
                    tab_hbm.at[idx_v.at[p]], rows, sems[j]).wait()
                for sub in range(PAIR):
                    accs0 = tuple(bias_v[pl.ds(c * LANES, LANES)]
                                  for c in range(EMB // LANES))

                    def body(r, accs):
                        new = list(accs)
                        for g in range(EMB // 32):
                            x = plsc.bitcast(
                                rows[r, pl.ds(LANES * g, LANES)], jnp.bfloat16)
                            a, b2 = plsc.unpack(
                                x, format=plsc.PackFormat.INTERLEAVED)
                            new[2 * g] = new[2 * g] + a
                            new[2 * g + 1] = new[2 * g + 1] + b2
                        return tuple(new)

                    accs = plsc.parallel_loop(
                        sub * MPD, (sub + 1) * MPD, 1, unroll=5,
                        carry=accs0)(body)
                    d = p * PAIR + sub
                    for c in range(EMB // LANES):
                        out_v[d, pl.ds(c * LANES, LANES)] = jnp.maximum(
                            accs[c], 0.0)

                @pl.when(p + NBUF < PAIRS_PER_W)
                def _():
                    pltpu.async_copy(
                        tab_hbm.at[idx_v.at[p + NBUF]], rows, sems[j])

        pltpu.sync_copy(out_v, out_hbm.at[pl.ds(base, DOCS_PER_W)])

    return kern(idx, table, bias)


def kernel(document_mention_indices, W, b):
    idx = document_mention_indices.astype(jnp.int32).reshape(
        BATCH // PAIR, PAIR * MPD)
    u = jax.lax.bitcast_convert_type(W.T, jnp.uint32).reshape(
        W.shape[1], EMB // 32, 2, 16)
    word = _bf16_bits(u[:, :, 0, :]) | (_bf16_bits(u[:, :, 1, :]) << 16)
    table = jax.lax.bitcast_convert_type(
        word.reshape(W.shape[1], EMB // 2), jnp.int32)
    return _sc_embedding_bag(idx, table, b)


# speedup vs baseline: 16.6054x; 5.5033x over previous
import functools

import jax
import jax.numpy as jnp
from jax import lax
from jax.experimental import pallas as pl
from jax.experimental.pallas import tpu as pltpu
from jax.experimental.pallas import tpu_sc as plsc

BATCH = 1024
MPD = 50
EMB = 128
LANES = 16
NC, NS = 2, 16
NW = NC * NS
DOCS_PER_W = BATCH // NW
PAIR = 2
PAIRS_PER_W = DOCS_PER_W // PAIR
NBUF = 4


def _sc_embedding_bag(idx, table, bias):
    mesh = plsc.VectorSubcoreMesh(core_axis_name="c", subcore_axis_name="s")

    @functools.partial(
        pl.kernel,
        out_type=jax.ShapeDtypeStruct((BATCH, EMB), jnp.float32),
        mesh=mesh,
        scratch_types=[
            pltpu.VMEM((PAIRS_PER_W, PAIR * MPD), jnp.int32),
        ]
        + [pltpu.VMEM((PAIR * MPD, EMB), jnp.float32)] * NBUF
        + [
            pltpu.VMEM((DOCS_PER_W, EMB), jnp.float32),
            pltpu.VMEM((EMB,), jnp.float32),
        ]
        + [pltpu.SemaphoreType.DMA] * NBUF,
    )
    def kern(idx_hbm, tab_hbm, b_hbm, out_hbm, idx_v, *rest):
        rows_bufs = rest[:NBUF]
        out_v, bias_v = rest[NBUF], rest[NBUF + 1]
        sems = rest[NBUF + 2:]
        wid = lax.axis_index("s") * NC + lax.axis_index("c")
        base = wid * DOCS_PER_W
        pltpu.sync_copy(b_hbm, bias_v)
        pltpu.sync_copy(idx_hbm.at[pl.ds(wid * PAIRS_PER_W, PAIRS_PER_W)], idx_v)

        for j in range(NBUF):
            pltpu.async_copy(tab_hbm.at[idx_v.at[j]], rows_bufs[j], sems[j])

        @pl.loop(0, PAIRS_PER_W, step=NBUF)
        def _pair(p0):
            for j in range(NBUF):
                p = p0 + j
                rows = rows_bufs[j]
                pltpu.make_async_copy(
                    tab_hbm.at[idx_v.at[p]], rows, sems[j]).wait()
                for sub in range(PAIR):
                    accs0 = tuple(bias_v[pl.ds(c * LANES, LANES)]
                                  for c in range(EMB // LANES))

                    def body(r, accs):
                        return tuple(
                            accs[c] + rows[r, pl.ds(c * LANES, LANES)]
                            for c in range(EMB // LANES))

                    accs = plsc.parallel_loop(
                        sub * MPD, (sub + 1) * MPD, 1, unroll=5,
                        carry=accs0)(body)
                    d = p * PAIR + sub
                    for c in range(EMB // LANES):
                        out_v[d, pl.ds(c * LANES, LANES)] = jnp.maximum(
                            accs[c], 0.0)

                @pl.when(p + NBUF < PAIRS_PER_W)
                def _():
                    pltpu.async_copy(
                        tab_hbm.at[idx_v.at[p + NBUF]], rows, sems[j])

        pltpu.sync_copy(out_v, out_hbm.at[pl.ds(base, DOCS_PER_W)])

    return kern(idx, table, bias)


def kernel(document_mention_indices, W, b):
    idx = document_mention_indices.astype(jnp.int32).reshape(
        BATCH // PAIR, PAIR * MPD)
    table = W.T
    return _sc_embedding_bag(idx, table, b)
